# baseline (speedup 1.0000x reference)
import functools
import math

import jax
import jax.numpy as jnp
from jax import lax
from jax.experimental import pallas as pl
from jax.experimental.pallas import tpu as pltpu
from jax.experimental.pallas import tpu_sc as plsc

NC, NS, LANES = 2, 16, 16
NW = NC * NS
CH = 128
ZR = 32


def _rpad(r, sck):
    q = CH * NW * sck
    return ((r + q - 1) // q) * q


def _nacc(n):
    q = NS * ZR
    return ((n + 1 + q - 1) // q) * q


def _blockdiag(w, rep):
    cin, cout = w.shape
    eye = jnp.eye(rep, dtype=w.dtype)
    return (eye[:, None, :, None] * w[None, :, None, :]).reshape(
        rep * cin, rep * cout
    )


def _tc_matmul(x, w, b2d):
    def body(x_ref, w_ref, b_ref, o_ref):
        o_ref[...] = (
            jnp.dot(x_ref[...], w_ref[...], preferred_element_type=jnp.float32)
            + b_ref[...]
        )
    return pl.pallas_call(
        body,
        out_shape=jax.ShapeDtypeStruct((x.shape[0], w.shape[1]), jnp.float32),
    )(x, w, b2d)


def _tc_combine_matmul(parts, b_in2d, w, b2d):
    def body(p_ref, bi_ref, w_ref, b_ref, o_ref):
        h = jnp.maximum(p_ref[0] + p_ref[1] + bi_ref[...], 0.0)
        o_ref[...] = (
            jnp.dot(h, w_ref[...], preferred_element_type=jnp.float32) + b_ref[...]
        )
    return pl.pallas_call(
        body,
        out_shape=jax.ShapeDtypeStruct((parts.shape[1], w.shape[1]), jnp.float32),
    )(parts, b_in2d, w, b2d)


def _tc_edge_matmul(gp, wbd, ep, rp_rows):
    kc, cw, _ = wbd.shape

    def body(g_ref, w_ref, o_ref):
        o_ref[...] = jnp.dot(
            g_ref[...], w_ref[0], preferred_element_type=jnp.float32
        )

    return pl.pallas_call(
        body,
        grid=(kc,),
        in_specs=[
            pl.BlockSpec((ep, cw), lambda k: (k, 0)),
            pl.BlockSpec((1, cw, 128), lambda k: (k, 0, 0)),
        ],
        out_specs=pl.BlockSpec((ep, 128), lambda k: (k, 0)),
        out_shape=jax.ShapeDtypeStruct((rp_rows, 128), jnp.float32),
    )(gp, wbd)


def _tc_final_combine(parts_r, btile):
    def body(p_ref, b_ref, o_ref):
        o_ref[...] = p_ref[0] + p_ref[1] + b_ref[...]
    return pl.pallas_call(
        body,
        out_shape=jax.ShapeDtypeStruct(parts_r.shape[1:], jnp.float32),
    )(parts_r, btile)


def _sc_gather(children, src_flat, prune, n_chunks_w, sck):
    r_pad = src_flat.shape[0]
    c = children.shape[1]
    m = prune.shape[0]
    mesh = plsc.VectorSubcoreMesh(core_axis_name="c", subcore_axis_name="s")

    @functools.partial(
        pl.kernel,
        out_type=jax.ShapeDtypeStruct((r_pad, c), jnp.float32),
        mesh=mesh,
        scratch_types=[
            pltpu.VMEM((m,), jnp.int32),
            pltpu.VMEM((sck * CH,), jnp.int32),
            pltpu.VMEM((sck, CH), jnp.int32),
            pltpu.VMEM((sck * CH, c), jnp.float32),
        ] + [pltpu.SemaphoreType.DMA] * sck,
        compiler_params=pltpu.CompilerParams(needs_layout_passes=False, use_tc_tiling_on_sc=False),
    )
    def kfn(ch_hbm, src_hbm, prune_hbm, g_hbm, prune_v, src_v, cidx_v, rows_v,
            *sems):
        cid = lax.axis_index("c")
        sid = lax.axis_index("s")
        wid = sid * NC + cid
        r0w = wid * n_chunks_w * CH
        pltpu.sync_copy(prune_hbm, prune_v)

        def body(i, carry):
            base = r0w + i * (sck * CH)
            pltpu.sync_copy(src_hbm.at[pl.ds(base, sck * CH)], src_v)
            for j in range(sck):
                for l in range(CH // LANES):
                    idx = src_v[pl.ds(j * CH + l * LANES, LANES)]
                    cidx_v[j, pl.ds(l * LANES, LANES)] = plsc.load_gather(
                        prune_v, [idx]
                    )
            descs = [
                pltpu.async_copy(
                    ch_hbm.at[cidx_v.at[j]],
                    rows_v.at[pl.ds(j * CH, CH)],
                    sems[j],
                )
                for j in range(sck)
            ]
            for d in descs:
                d.wait()
            pltpu.sync_copy(rows_v, g_hbm.at[pl.ds(base, sck * CH)])
            return carry

        lax.fori_loop(0, n_chunks_w // sck, body, 0)

    return kfn(children, src_flat, prune)


def _sc_scatter(m_rows, dst_chunks, n_out, n_chunks_w, sck):
    cp = m_rows.shape[1]
    n_acc = _nacc(n_out)
    nz = n_acc // NS
    nzc = nz // ZR
    mesh = plsc.VectorSubcoreMesh(core_axis_name="c", subcore_axis_name="s")

    @functools.partial(
        pl.kernel,
        out_type=jax.ShapeDtypeStruct((NC, n_acc, cp), jnp.float32),
        mesh=mesh,
        scratch_types=[
            pltpu.VMEM((sck * CH, cp), jnp.float32),
            pltpu.VMEM((sck, CH), jnp.int32),
            pltpu.VMEM((ZR, cp), jnp.float32),
            pltpu.VMEM_SHARED((n_acc, cp), jnp.float32),
        ] + [pltpu.SemaphoreType.DMA] * sck,
        compiler_params=pltpu.CompilerParams(needs_layout_passes=False, use_tc_tiling_on_sc=False),
    )
    def kfn(m_hbm, dst_hbm, out_hbm, rows_v, dst_v, zero_v, acc, *sems):
        cid = lax.axis_index("c")
        sid = lax.axis_index("s")
        wid = sid * NC + cid
        c0w = wid * n_chunks_w

        def zfill(i, carry):
            def zlane(j, carry2):
                zero_v[i, pl.ds(j * LANES, LANES)] = jnp.zeros(
                    (LANES,), jnp.float32
                )
                return carry2
            lax.fori_loop(0, cp // LANES, zlane, 0)
            return carry

        lax.fori_loop(0, ZR, zfill, 0)

        def zacc(i, carry):
            pltpu.sync_copy(zero_v, acc.at[pl.ds(sid * nz + i * ZR, ZR)])
            return carry

        lax.fori_loop(0, nzc, zacc, 0)
        plsc.subcore_barrier()

        def body(i, carry):
            base = c0w + i * sck
            pltpu.sync_copy(m_hbm.at[pl.ds(base * CH, sck * CH)], rows_v)
            pltpu.sync_copy(dst_hbm.at[pl.ds(base, sck)], dst_v)
            descs = [
                pltpu.async_copy(
                    rows_v.at[pl.ds(j * CH, CH)],
                    acc.at[dst_v.at[j]],
                    sems[j],
                    add=True,
                )
                for j in range(sck)
            ]
            for d in descs:
                d.wait()
            return carry

        lax.fori_loop(0, n_chunks_w // sck, body, 0)
        plsc.subcore_barrier()
        pltpu.sync_copy(
            acc.at[pl.ds(sid * nz, nz)], out_hbm.at[cid].at[pl.ds(sid * nz, nz)]
        )

    return kfn(m_rows, dst_chunks)


def _conv_level(children, prune, src, dst, w_blk, n_out, sck_g, sck_s):
    kc, e = src.shape
    c = children.shape[1]
    cout = w_blk.shape[-1]
    sck_lcm = sck_g * sck_s // math.gcd(sck_g, sck_s)
    r = kc * e
    r_pad = _rpad(r, sck_lcm)
    n_chunks_w = r_pad // (CH * NW)
    nch = r_pad // CH

    srcf = jnp.pad(src.reshape(r), (0, r_pad - r))
    dstf = jnp.pad(dst.reshape(r), (0, r_pad - r), constant_values=n_out)

    g = _sc_gather(children, srcf, prune, n_chunks_w, sck_g)

    pk = 128 // cout
    cw = pk * c
    wbd = jnp.stack([_blockdiag(w_blk[k], pk) for k in range(kc)])
    m_packed = _tc_edge_matmul(
        g.reshape(r_pad // pk, cw), wbd, e // pk, r_pad // pk
    )

    return _sc_scatter(
        m_packed.reshape(r_pad, cout), dstf.reshape(nch, CH), n_out,
        n_chunks_w, sck_s,
    )


def kernel(x, W_up1, b_up1, W_blk1, b_blk1, W_up2, b_up2, W_blk2, b_blk2,
           W_up3, b_up3, W_blk3, b_blk3, prune1_idx, prune2_idx, prune3_idx,
           src1, dst1, src2, dst2, src3, dst3):
    kup = W_up1.shape[0]

    def upw(w):
        return jnp.transpose(w, (1, 0, 2)).reshape(w.shape[1], -1)

    def upb(b):
        return jnp.tile(b, (kup,)).reshape(1, -1)

    n2, n1, n0 = prune1_idx.shape[0], prune2_idx.shape[0], prune3_idx.shape[0]

    ch1 = _tc_matmul(x, upw(W_up1), upb(b_up1))
    ch1 = ch1.reshape(-1, W_up1.shape[-1])
    p1 = _conv_level(ch1, prune1_idx, src1, dst1, W_blk1, n2, 2, 4)

    ch2 = _tc_combine_matmul(
        p1, b_blk1.reshape(1, -1), upw(W_up2), upb(b_up2)
    )
    ch2 = ch2.reshape(-1, W_up2.shape[-1])
    p2 = _conv_level(ch2, prune2_idx, src2, dst2, W_blk2, n1, 2, 4)

    n_acc1 = p2.shape[1]
    c2 = W_up3.shape[1]
    p2pk = p2.reshape(NC, n_acc1 // 2, 2 * c2)
    bi2 = jnp.tile(b_blk2, (2,)).reshape(1, -1)
    w3bd = _blockdiag(upw(W_up3), 2)
    b3t = jnp.tile(jnp.tile(b_up3, (kup,)), (2,)).reshape(1, -1)
    ch3 = _tc_combine_matmul(p2pk, bi2, w3bd, b3t)
    ch3 = ch3.reshape(-1, W_up3.shape[-1])

    cout = W_blk3.shape[-1]
    cpad = LANES
    w3p = jnp.pad(W_blk3, ((0, 0), (0, 0), (0, cpad - cout)))
    p3 = _conv_level(ch3, prune3_idx, src3, dst3, w3p, n0, 2, 16)

    n_acc0 = p3.shape[1]
    parts_r = p3.reshape(NC, n_acc0 * cpad // 128, 128)
    b3tile = jnp.tile(jnp.pad(b_blk3, (0, cpad - cout)), (128 // cpad,)).reshape(1, 128)
    outr = _tc_final_combine(parts_r, b3tile)
    return outr.reshape(n_acc0, cpad)[:n0, :cout]


# speedup vs baseline: 1.0541x; 1.0541x over previous
import functools
import math

import jax
import jax.numpy as jnp
from jax import lax
from jax.experimental import pallas as pl
from jax.experimental.pallas import tpu as pltpu
from jax.experimental.pallas import tpu_sc as plsc

NC, NS, LANES = 2, 16, 16
NW = NC * NS
CH = 128
ZR = 32


def _rpad(r, sck):
    q = CH * NW * sck
    return ((r + q - 1) // q) * q


def _nacc(n):
    q = NS * ZR
    return ((n + 1 + q - 1) // q) * q


def _blockdiag(w, rep):
    cin, cout = w.shape
    eye = jnp.eye(rep, dtype=w.dtype)
    return (eye[:, None, :, None] * w[None, :, None, :]).reshape(
        rep * cin, rep * cout
    )


def _tc_matmul(x, w, b2d):
    def body(x_ref, w_ref, b_ref, o_ref):
        o_ref[...] = (
            jnp.dot(x_ref[...], w_ref[...], preferred_element_type=jnp.float32)
            + b_ref[...]
        )
    return pl.pallas_call(
        body,
        out_shape=jax.ShapeDtypeStruct((x.shape[0], w.shape[1]), jnp.float32),
    )(x, w, b2d)


def _tc_combine_matmul(parts, b_in2d, w, b2d):
    def body(p_ref, bi_ref, w_ref, b_ref, o_ref):
        h = jnp.maximum(p_ref[0] + p_ref[1] + bi_ref[...], 0.0)
        o_ref[...] = (
            jnp.dot(h, w_ref[...], preferred_element_type=jnp.float32) + b_ref[...]
        )
    return pl.pallas_call(
        body,
        out_shape=jax.ShapeDtypeStruct((parts.shape[1], w.shape[1]), jnp.float32),
    )(parts, b_in2d, w, b2d)


def _tc_edge_matmul(gp, wbd, ep, rp_rows):
    kc, cw, _ = wbd.shape

    def body(g_ref, w_ref, o_ref):
        o_ref[...] = jnp.dot(
            g_ref[...], w_ref[0], preferred_element_type=jnp.float32
        )

    return pl.pallas_call(
        body,
        grid=(kc,),
        in_specs=[
            pl.BlockSpec((ep, cw), lambda k: (k, 0)),
            pl.BlockSpec((1, cw, 128), lambda k: (k, 0, 0)),
        ],
        out_specs=pl.BlockSpec((ep, 128), lambda k: (k, 0)),
        out_shape=jax.ShapeDtypeStruct((rp_rows, 128), jnp.float32),
    )(gp, wbd)


def _tc_final_combine(parts_r, btile):
    def body(p_ref, b_ref, o_ref):
        o_ref[...] = p_ref[0] + p_ref[1] + b_ref[...]
    return pl.pallas_call(
        body,
        out_shape=jax.ShapeDtypeStruct(parts_r.shape[1:], jnp.float32),
    )(parts_r, btile)


def _sc_gather(children, src_flat, prune, n_chunks_w, sck):
    r_pad = src_flat.shape[0]
    c = children.shape[1]
    m = prune.shape[0]
    mesh = plsc.VectorSubcoreMesh(core_axis_name="c", subcore_axis_name="s")

    @functools.partial(
        pl.kernel,
        out_type=jax.ShapeDtypeStruct((r_pad, c), jnp.float32),
        mesh=mesh,
        scratch_types=[
            pltpu.VMEM((m,), jnp.int32),
            pltpu.VMEM((sck * CH,), jnp.int32),
            pltpu.VMEM((sck, CH), jnp.int32),
            pltpu.VMEM((sck * CH, c), jnp.float32),
        ] + [pltpu.SemaphoreType.DMA] * sck,
        compiler_params=pltpu.CompilerParams(needs_layout_passes=False, use_tc_tiling_on_sc=False),
    )
    def kfn(ch_hbm, src_hbm, prune_hbm, g_hbm, prune_v, src_v, cidx_v, rows_v,
            *sems):
        cid = lax.axis_index("c")
        sid = lax.axis_index("s")
        wid = sid * NC + cid
        r0w = wid * n_chunks_w * CH
        pltpu.sync_copy(prune_hbm, prune_v)

        def body(i, carry):
            base = r0w + i * (sck * CH)
            pltpu.sync_copy(src_hbm.at[pl.ds(base, sck * CH)], src_v)
            for j in range(sck):
                for l in range(CH // LANES):
                    idx = src_v[pl.ds(j * CH + l * LANES, LANES)]
                    cidx_v[j, pl.ds(l * LANES, LANES)] = plsc.load_gather(
                        prune_v, [idx]
                    )
            descs = [
                pltpu.async_copy(
                    ch_hbm.at[cidx_v.at[j]],
                    rows_v.at[pl.ds(j * CH, CH)],
                    sems[j],
                )
                for j in range(sck)
            ]
            for d in descs:
                d.wait()
            pltpu.sync_copy(rows_v, g_hbm.at[pl.ds(base, sck * CH)])
            return carry

        lax.fori_loop(0, n_chunks_w // sck, body, 0)

    return kfn(children, src_flat, prune)


def _sc_scatter(m_rows, dst_chunks, n_out, n_chunks_w, sck):
    cp = m_rows.shape[1]
    n_acc = _nacc(n_out)
    nz = n_acc // NS
    nzc = nz // ZR
    mesh = plsc.VectorSubcoreMesh(core_axis_name="c", subcore_axis_name="s")

    @functools.partial(
        pl.kernel,
        out_type=jax.ShapeDtypeStruct((NC, n_acc, cp), jnp.float32),
        mesh=mesh,
        scratch_types=[
            pltpu.VMEM((sck * CH, cp), jnp.float32),
            pltpu.VMEM((sck, CH), jnp.int32),
            pltpu.VMEM((ZR, cp), jnp.float32),
            pltpu.VMEM_SHARED((n_acc, cp), jnp.float32),
        ] + [pltpu.SemaphoreType.DMA] * sck,
        compiler_params=pltpu.CompilerParams(needs_layout_passes=False, use_tc_tiling_on_sc=False),
    )
    def kfn(m_hbm, dst_hbm, out_hbm, rows_v, dst_v, zero_v, acc, *sems):
        cid = lax.axis_index("c")
        sid = lax.axis_index("s")
        wid = sid * NC + cid
        c0w = wid * n_chunks_w

        def zfill(i, carry):
            def zlane(j, carry2):
                zero_v[i, pl.ds(j * LANES, LANES)] = jnp.zeros(
                    (LANES,), jnp.float32
                )
                return carry2
            lax.fori_loop(0, cp // LANES, zlane, 0)
            return carry

        lax.fori_loop(0, ZR, zfill, 0)

        def zacc(i, carry):
            pltpu.sync_copy(zero_v, acc.at[pl.ds(sid * nz + i * ZR, ZR)])
            return carry

        lax.fori_loop(0, nzc, zacc, 0)
        plsc.subcore_barrier()

        def body(i, carry):
            base = c0w + i * sck
            pltpu.sync_copy(m_hbm.at[pl.ds(base * CH, sck * CH)], rows_v)
            pltpu.sync_copy(dst_hbm.at[pl.ds(base, sck)], dst_v)
            descs = [
                pltpu.async_copy(
                    rows_v.at[pl.ds(j * CH, CH)],
                    acc.at[dst_v.at[j]],
                    sems[j],
                    add=True,
                )
                for j in range(sck)
            ]
            for d in descs:
                d.wait()
            return carry

        lax.fori_loop(0, n_chunks_w // sck, body, 0)
        plsc.subcore_barrier()
        pltpu.sync_copy(
            acc.at[pl.ds(sid * nz, nz)], out_hbm.at[cid].at[pl.ds(sid * nz, nz)]
        )

    return kfn(m_rows, dst_chunks)


def _conv_level(children, prune, src, dst, w_blk, n_out, sck_g, sck_s):
    kc, e = src.shape
    c = children.shape[1]
    cout = w_blk.shape[-1]
    sck_lcm = sck_g * sck_s // math.gcd(sck_g, sck_s)
    r = kc * e
    r_pad = _rpad(r, sck_lcm)
    n_chunks_w = r_pad // (CH * NW)
    nch = r_pad // CH

    srcf = jnp.pad(src.reshape(r), (0, r_pad - r))
    dstf = jnp.pad(dst.reshape(r), (0, r_pad - r), constant_values=n_out)

    g = _sc_gather(children, srcf, prune, n_chunks_w, sck_g)

    pk = 128 // cout
    cw = pk * c
    wbd = jnp.stack([_blockdiag(w_blk[k], pk) for k in range(kc)])
    m_packed = _tc_edge_matmul(
        g.reshape(r_pad // pk, cw), wbd, e // pk, r_pad // pk
    )

    return _sc_scatter(
        m_packed.reshape(r_pad, cout), dstf.reshape(nch, CH), n_out,
        n_chunks_w, sck_s,
    )


def kernel(x, W_up1, b_up1, W_blk1, b_blk1, W_up2, b_up2, W_blk2, b_blk2,
           W_up3, b_up3, W_blk3, b_blk3, prune1_idx, prune2_idx, prune3_idx,
           src1, dst1, src2, dst2, src3, dst3):
    kup = W_up1.shape[0]

    def upw(w):
        return jnp.transpose(w, (1, 0, 2)).reshape(w.shape[1], -1)

    def upb(b):
        return jnp.tile(b, (kup,)).reshape(1, -1)

    n2, n1, n0 = prune1_idx.shape[0], prune2_idx.shape[0], prune3_idx.shape[0]

    ch1 = _tc_matmul(x, upw(W_up1), upb(b_up1))
    ch1 = ch1.reshape(-1, W_up1.shape[-1])
    p1 = _conv_level(ch1, prune1_idx, src1, dst1, W_blk1, n2, 4, 4)

    ch2 = _tc_combine_matmul(
        p1, b_blk1.reshape(1, -1), upw(W_up2), upb(b_up2)
    )
    ch2 = ch2.reshape(-1, W_up2.shape[-1])
    p2 = _conv_level(ch2, prune2_idx, src2, dst2, W_blk2, n1, 4, 4)

    n_acc1 = p2.shape[1]
    c2 = W_up3.shape[1]
    p2pk = p2.reshape(NC, n_acc1 // 2, 2 * c2)
    bi2 = jnp.tile(b_blk2, (2,)).reshape(1, -1)
    w3bd = _blockdiag(upw(W_up3), 2)
    b3t = jnp.tile(jnp.tile(b_up3, (kup,)), (2,)).reshape(1, -1)
    ch3 = _tc_combine_matmul(p2pk, bi2, w3bd, b3t)
    ch3 = ch3.reshape(-1, W_up3.shape[-1])

    cout = W_blk3.shape[-1]
    cpad = LANES
    w3p = jnp.pad(W_blk3, ((0, 0), (0, 0), (0, cpad - cout)))
    p3 = _conv_level(ch3, prune3_idx, src3, dst3, w3p, n0, 16, 16)

    n_acc0 = p3.shape[1]
    parts_r = p3.reshape(NC, n_acc0 * cpad // 128, 128)
    b3tile = jnp.tile(jnp.pad(b_blk3, (0, cpad - cout)), (128 // cpad,)).reshape(1, 128)
    outr = _tc_final_combine(parts_r, b3tile)
    return outr.reshape(n_acc0, cpad)[:n0, :cout]


# speedup vs baseline: 1.2697x; 1.2045x over previous
import functools
import math

import jax
import jax.numpy as jnp
from jax import lax
from jax.experimental import pallas as pl
from jax.experimental.pallas import tpu as pltpu
from jax.experimental.pallas import tpu_sc as plsc

NC, NS, LANES = 2, 16, 16
NW = NC * NS
CH = 128
ZR = 32


def _rpad(r, sck):
    q = CH * NW * sck
    return ((r + q - 1) // q) * q


def _nacc(n):
    q = NS * ZR
    return ((n + 1 + q - 1) // q) * q


def _blockdiag(w, rep):
    cin, cout = w.shape
    eye = jnp.eye(rep, dtype=w.dtype)
    return (eye[:, None, :, None] * w[None, :, None, :]).reshape(
        rep * cin, rep * cout
    )


def _tc_matmul(x, w, b2d):
    def body(x_ref, w_ref, b_ref, o_ref):
        o_ref[...] = (
            jnp.dot(x_ref[...], w_ref[...], preferred_element_type=jnp.float32)
            + b_ref[...]
        )
    return pl.pallas_call(
        body,
        out_shape=jax.ShapeDtypeStruct((x.shape[0], w.shape[1]), jnp.float32),
    )(x, w, b2d)


def _tc_combine_matmul(parts, b_in2d, w, b2d):
    def body(p_ref, bi_ref, w_ref, b_ref, o_ref):
        h = jnp.maximum(p_ref[0] + p_ref[1] + bi_ref[...], 0.0)
        o_ref[...] = (
            jnp.dot(h, w_ref[...], preferred_element_type=jnp.float32) + b_ref[...]
        )
    return pl.pallas_call(
        body,
        out_shape=jax.ShapeDtypeStruct((parts.shape[1], w.shape[1]), jnp.float32),
    )(parts, b_in2d, w, b2d)


def _tc_edge_matmul(gp, wbd, ep, rp_rows):
    kc, cw, _ = wbd.shape

    def body(g_ref, w_ref, o_ref):
        o_ref[...] = jnp.dot(
            g_ref[...], w_ref[0], preferred_element_type=jnp.float32
        )

    return pl.pallas_call(
        body,
        grid=(kc,),
        in_specs=[
            pl.BlockSpec((ep, cw), lambda k: (k, 0)),
            pl.BlockSpec((1, cw, 128), lambda k: (k, 0, 0)),
        ],
        out_specs=pl.BlockSpec((ep, 128), lambda k: (k, 0)),
        out_shape=jax.ShapeDtypeStruct((rp_rows, 128), jnp.float32),
    )(gp, wbd)


def _tc_final_combine(parts_r, btile):
    def body(p_ref, b_ref, o_ref):
        o_ref[...] = p_ref[0] + p_ref[1] + b_ref[...]
    return pl.pallas_call(
        body,
        out_shape=jax.ShapeDtypeStruct(parts_r.shape[1:], jnp.float32),
    )(parts_r, btile)


def _sc_gather(children, src_flat, prune, n_chunks_w, sck, stage=False):
    r_pad = src_flat.shape[0]
    rows8n = children.shape[0]
    c = children.shape[1]
    m = prune.shape[0]
    mesh = plsc.VectorSubcoreMesh(core_axis_name="c", subcore_axis_name="s")
    shared_types = (
        [pltpu.VMEM_SHARED((rows8n, c), jnp.float32)] if stage else []
    )

    @functools.partial(
        pl.kernel,
        out_type=jax.ShapeDtypeStruct((r_pad, c), jnp.float32),
        mesh=mesh,
        scratch_types=[
            pltpu.VMEM((m,), jnp.int32),
            pltpu.VMEM((sck * CH,), jnp.int32),
            pltpu.VMEM((sck, CH), jnp.int32),
            pltpu.VMEM((sck * CH, c), jnp.float32),
        ] + shared_types + [pltpu.SemaphoreType.DMA] * sck,
        compiler_params=pltpu.CompilerParams(needs_layout_passes=False, use_tc_tiling_on_sc=False),
    )
    def kfn(ch_hbm, src_hbm, prune_hbm, g_hbm, prune_v, src_v, cidx_v, rows_v,
            *shared_and_sems):
        if stage:
            table = shared_and_sems[0]
            sems = shared_and_sems[1:]
        else:
            table = ch_hbm
            sems = shared_and_sems
        cid = lax.axis_index("c")
        sid = lax.axis_index("s")
        wid = sid * NC + cid
        r0w = wid * n_chunks_w * CH
        pltpu.sync_copy(prune_hbm, prune_v)
        if stage:
            nst = rows8n // NS
            pltpu.sync_copy(
                ch_hbm.at[pl.ds(sid * nst, nst)], table.at[pl.ds(sid * nst, nst)]
            )
            plsc.subcore_barrier()

        def body(i, carry):
            base = r0w + i * (sck * CH)
            pltpu.sync_copy(src_hbm.at[pl.ds(base, sck * CH)], src_v)
            for j in range(sck):
                for l in range(CH // LANES):
                    idx = src_v[pl.ds(j * CH + l * LANES, LANES)]
                    cidx_v[j, pl.ds(l * LANES, LANES)] = plsc.load_gather(
                        prune_v, [idx]
                    )
            descs = [
                pltpu.async_copy(
                    table.at[cidx_v.at[j]],
                    rows_v.at[pl.ds(j * CH, CH)],
                    sems[j],
                )
                for j in range(sck)
            ]
            for d in descs:
                d.wait()
            pltpu.sync_copy(rows_v, g_hbm.at[pl.ds(base, sck * CH)])
            return carry

        lax.fori_loop(0, n_chunks_w // sck, body, 0)

    return kfn(children, src_flat, prune)


def _sc_scatter(m_rows, dst_chunks, n_out, n_chunks_w, sck):
    cp = m_rows.shape[1]
    n_acc = _nacc(n_out)
    nz = n_acc // NS
    nzc = nz // ZR
    mesh = plsc.VectorSubcoreMesh(core_axis_name="c", subcore_axis_name="s")

    @functools.partial(
        pl.kernel,
        out_type=jax.ShapeDtypeStruct((NC, n_acc, cp), jnp.float32),
        mesh=mesh,
        scratch_types=[
            pltpu.VMEM((sck * CH, cp), jnp.float32),
            pltpu.VMEM((sck, CH), jnp.int32),
            pltpu.VMEM((ZR, cp), jnp.float32),
            pltpu.VMEM_SHARED((n_acc, cp), jnp.float32),
        ] + [pltpu.SemaphoreType.DMA] * sck,
        compiler_params=pltpu.CompilerParams(needs_layout_passes=False, use_tc_tiling_on_sc=False),
    )
    def kfn(m_hbm, dst_hbm, out_hbm, rows_v, dst_v, zero_v, acc, *sems):
        cid = lax.axis_index("c")
        sid = lax.axis_index("s")
        wid = sid * NC + cid
        c0w = wid * n_chunks_w

        def zfill(i, carry):
            def zlane(j, carry2):
                zero_v[i, pl.ds(j * LANES, LANES)] = jnp.zeros(
                    (LANES,), jnp.float32
                )
                return carry2
            lax.fori_loop(0, cp // LANES, zlane, 0)
            return carry

        lax.fori_loop(0, ZR, zfill, 0)

        def zacc(i, carry):
            pltpu.sync_copy(zero_v, acc.at[pl.ds(sid * nz + i * ZR, ZR)])
            return carry

        lax.fori_loop(0, nzc, zacc, 0)
        plsc.subcore_barrier()

        def body(i, carry):
            base = c0w + i * sck
            pltpu.sync_copy(m_hbm.at[pl.ds(base * CH, sck * CH)], rows_v)
            pltpu.sync_copy(dst_hbm.at[pl.ds(base, sck)], dst_v)
            descs = [
                pltpu.async_copy(
                    rows_v.at[pl.ds(j * CH, CH)],
                    acc.at[dst_v.at[j]],
                    sems[j],
                    add=True,
                )
                for j in range(sck)
            ]
            for d in descs:
                d.wait()
            return carry

        lax.fori_loop(0, n_chunks_w // sck, body, 0)
        plsc.subcore_barrier()
        pltpu.sync_copy(
            acc.at[pl.ds(sid * nz, nz)], out_hbm.at[cid].at[pl.ds(sid * nz, nz)]
        )

    return kfn(m_rows, dst_chunks)


def _conv_level(children, prune, src, dst, w_blk, n_out, sck_g, sck_s,
                stage=False):
    kc, e = src.shape
    c = children.shape[1]
    cout = w_blk.shape[-1]
    sck_lcm = sck_g * sck_s // math.gcd(sck_g, sck_s)
    r = kc * e
    r_pad = _rpad(r, sck_lcm)
    n_chunks_w = r_pad // (CH * NW)
    nch = r_pad // CH

    srcf = jnp.pad(src.reshape(r), (0, r_pad - r))
    dstf = jnp.pad(dst.reshape(r), (0, r_pad - r), constant_values=n_out)

    g = _sc_gather(children, srcf, prune, n_chunks_w, sck_g, stage)

    pk = 128 // cout
    cw = pk * c
    wbd = jnp.stack([_blockdiag(w_blk[k], pk) for k in range(kc)])
    m_packed = _tc_edge_matmul(
        g.reshape(r_pad // pk, cw), wbd, e // pk, r_pad // pk
    )

    return _sc_scatter(
        m_packed.reshape(r_pad, cout), dstf.reshape(nch, CH), n_out,
        n_chunks_w, sck_s,
    )


def kernel(x, W_up1, b_up1, W_blk1, b_blk1, W_up2, b_up2, W_blk2, b_blk2,
           W_up3, b_up3, W_blk3, b_blk3, prune1_idx, prune2_idx, prune3_idx,
           src1, dst1, src2, dst2, src3, dst3):
    kup = W_up1.shape[0]

    def upw(w):
        return jnp.transpose(w, (1, 0, 2)).reshape(w.shape[1], -1)

    def upb(b):
        return jnp.tile(b, (kup,)).reshape(1, -1)

    n2, n1, n0 = prune1_idx.shape[0], prune2_idx.shape[0], prune3_idx.shape[0]

    ch1 = _tc_matmul(x, upw(W_up1), upb(b_up1))
    ch1 = ch1.reshape(-1, W_up1.shape[-1])
    p1 = _conv_level(ch1, prune1_idx, src1, dst1, W_blk1, n2, 4, 4, stage=True)

    ch2 = _tc_combine_matmul(
        p1, b_blk1.reshape(1, -1), upw(W_up2), upb(b_up2)
    )
    ch2 = ch2.reshape(-1, W_up2.shape[-1])
    p2 = _conv_level(ch2, prune2_idx, src2, dst2, W_blk2, n1, 4, 4)

    n_acc1 = p2.shape[1]
    c2 = W_up3.shape[1]
    p2pk = p2.reshape(NC, n_acc1 // 2, 2 * c2)
    bi2 = jnp.tile(b_blk2, (2,)).reshape(1, -1)
    w3bd = _blockdiag(upw(W_up3), 2)
    b3t = jnp.tile(jnp.tile(b_up3, (kup,)), (2,)).reshape(1, -1)
    ch3 = _tc_combine_matmul(p2pk, bi2, w3bd, b3t)
    ch3 = ch3.reshape(-1, W_up3.shape[-1])

    cout = W_blk3.shape[-1]
    cpad = LANES
    w3p = jnp.pad(W_blk3, ((0, 0), (0, 0), (0, cpad - cout)))
    p3 = _conv_level(ch3, prune3_idx, src3, dst3, w3p, n0, 16, 16)

    n_acc0 = p3.shape[1]
    parts_r = p3.reshape(NC, n_acc0 * cpad // 128, 128)
    b3tile = jnp.tile(jnp.pad(b_blk3, (0, cpad - cout)), (128 // cpad,)).reshape(1, 128)
    outr = _tc_final_combine(parts_r, b3tile)
    return outr.reshape(n_acc0, cpad)[:n0, :cout]


# speedup vs baseline: 1.4629x; 1.1521x over previous
import functools
import math

import jax
import jax.numpy as jnp
from jax import lax
from jax.experimental import pallas as pl
from jax.experimental.pallas import tpu as pltpu
from jax.experimental.pallas import tpu_sc as plsc

NC, NS, LANES = 2, 16, 16
NW = NC * NS
CH = 128
ZR = 32


def _rpad(r, sck):
    q = CH * NW * sck
    return ((r + q - 1) // q) * q


def _nacc(n):
    q = NS * ZR
    return ((n + 1 + q - 1) // q) * q


def _blockdiag(w, rep):
    cin, cout = w.shape
    eye = jnp.eye(rep, dtype=w.dtype)
    return (eye[:, None, :, None] * w[None, :, None, :]).reshape(
        rep * cin, rep * cout
    )


def _tc_matmul(x, w, b2d):
    def body(x_ref, w_ref, b_ref, o_ref):
        o_ref[...] = (
            jnp.dot(x_ref[...], w_ref[...], preferred_element_type=jnp.float32)
            + b_ref[...]
        )
    return pl.pallas_call(
        body,
        out_shape=jax.ShapeDtypeStruct((x.shape[0], w.shape[1]), jnp.float32),
    )(x, w, b2d)


def _tc_combine_matmul(parts, b_in2d, w, b2d):
    def body(p_ref, bi_ref, w_ref, b_ref, o_ref):
        h = jnp.maximum(p_ref[0] + p_ref[1] + bi_ref[...], 0.0)
        o_ref[...] = (
            jnp.dot(h, w_ref[...], preferred_element_type=jnp.float32) + b_ref[...]
        )
    return pl.pallas_call(
        body,
        out_shape=jax.ShapeDtypeStruct((parts.shape[1], w.shape[1]), jnp.float32),
    )(parts, b_in2d, w, b2d)


def _tc_edge_matmul(gp, wbd, ep, rp_rows):
    kc, cw, _ = wbd.shape

    def body(g_ref, w_ref, o_ref):
        o_ref[...] = jnp.dot(
            g_ref[...], w_ref[0], preferred_element_type=jnp.float32
        )

    return pl.pallas_call(
        body,
        grid=(kc,),
        in_specs=[
            pl.BlockSpec((ep, cw), lambda k: (k, 0)),
            pl.BlockSpec((1, cw, 128), lambda k: (k, 0, 0)),
        ],
        out_specs=pl.BlockSpec((ep, 128), lambda k: (k, 0)),
        out_shape=jax.ShapeDtypeStruct((rp_rows, 128), jnp.float32),
    )(gp, wbd)


def _tc_final_combine(parts_r, btile):
    def body(p_ref, b_ref, o_ref):
        o_ref[...] = p_ref[0] + p_ref[1] + b_ref[...]
    return pl.pallas_call(
        body,
        out_shape=jax.ShapeDtypeStruct(parts_r.shape[1:], jnp.float32),
    )(parts_r, btile)


def _sc_gather(children, src_flat, prune, n_chunks_w, sck, stage=False):
    r_pad = src_flat.shape[0]
    rows8n = children.shape[0]
    c = children.shape[1]
    m = prune.shape[0]
    mesh = plsc.VectorSubcoreMesh(core_axis_name="c", subcore_axis_name="s")
    st_rows = stage if stage else 0
    shared_types = (
        [pltpu.VMEM_SHARED((st_rows, c), jnp.float32)] if stage else []
    )

    @functools.partial(
        pl.kernel,
        out_type=jax.ShapeDtypeStruct((r_pad, c), jnp.float32),
        mesh=mesh,
        scratch_types=[
            pltpu.VMEM((m,), jnp.int32),
            pltpu.VMEM((sck * CH,), jnp.int32),
            pltpu.VMEM((sck, CH), jnp.int32),
            pltpu.VMEM((sck * CH, c), jnp.float32),
        ] + shared_types + [pltpu.SemaphoreType.DMA] * sck,
        compiler_params=pltpu.CompilerParams(needs_layout_passes=False, use_tc_tiling_on_sc=False),
    )
    def kfn(ch_hbm, src_hbm, prune_hbm, g_hbm, prune_v, src_v, cidx_v, rows_v,
            *shared_and_sems):
        if stage:
            table = shared_and_sems[0]
            sems = shared_and_sems[1:]
        else:
            table = ch_hbm
            sems = shared_and_sems
        cid = lax.axis_index("c")
        sid = lax.axis_index("s")
        wid = sid * NC + cid
        r0w = wid * n_chunks_w * CH
        pltpu.sync_copy(prune_hbm, prune_v)
        if stage:
            nst = st_rows // NS
            pltpu.sync_copy(
                ch_hbm.at[pl.ds(sid * nst, nst)], table.at[pl.ds(sid * nst, nst)]
            )
            plsc.subcore_barrier()

        def body(i, carry):
            base = r0w + i * (sck * CH)
            pltpu.sync_copy(src_hbm.at[pl.ds(base, sck * CH)], src_v)
            for j in range(sck):
                for l in range(CH // LANES):
                    idx = src_v[pl.ds(j * CH + l * LANES, LANES)]
                    cidx_v[j, pl.ds(l * LANES, LANES)] = plsc.load_gather(
                        prune_v, [idx]
                    )
            descs = [
                pltpu.async_copy(
                    table.at[cidx_v.at[j]],
                    rows_v.at[pl.ds(j * CH, CH)],
                    sems[j],
                )
                for j in range(sck)
            ]
            for d in descs:
                d.wait()
            pltpu.sync_copy(rows_v, g_hbm.at[pl.ds(base, sck * CH)])
            return carry

        lax.fori_loop(0, n_chunks_w // sck, body, 0)

    return kfn(children, src_flat, prune)


def _sc_scatter(m_rows, dst_chunks, n_out, n_chunks_w, sck):
    cp = m_rows.shape[1]
    n_acc = _nacc(n_out)
    nz = n_acc // NS
    nzc = nz // ZR
    mesh = plsc.VectorSubcoreMesh(core_axis_name="c", subcore_axis_name="s")

    @functools.partial(
        pl.kernel,
        out_type=jax.ShapeDtypeStruct((NC, n_acc, cp), jnp.float32),
        mesh=mesh,
        scratch_types=[
            pltpu.VMEM((sck * CH, cp), jnp.float32),
            pltpu.VMEM((sck, CH), jnp.int32),
            pltpu.VMEM((ZR, cp), jnp.float32),
            pltpu.VMEM_SHARED((n_acc, cp), jnp.float32),
        ] + [pltpu.SemaphoreType.DMA] * sck,
        compiler_params=pltpu.CompilerParams(needs_layout_passes=False, use_tc_tiling_on_sc=False),
    )
    def kfn(m_hbm, dst_hbm, out_hbm, rows_v, dst_v, zero_v, acc, *sems):
        cid = lax.axis_index("c")
        sid = lax.axis_index("s")
        wid = sid * NC + cid
        c0w = wid * n_chunks_w

        def zfill(i, carry):
            def zlane(j, carry2):
                zero_v[i, pl.ds(j * LANES, LANES)] = jnp.zeros(
                    (LANES,), jnp.float32
                )
                return carry2
            lax.fori_loop(0, cp // LANES, zlane, 0)
            return carry

        lax.fori_loop(0, ZR, zfill, 0)

        def zacc(i, carry):
            pltpu.sync_copy(zero_v, acc.at[pl.ds(sid * nz + i * ZR, ZR)])
            return carry

        lax.fori_loop(0, nzc, zacc, 0)
        plsc.subcore_barrier()

        def body(i, carry):
            base = c0w + i * sck
            pltpu.sync_copy(m_hbm.at[pl.ds(base * CH, sck * CH)], rows_v)
            pltpu.sync_copy(dst_hbm.at[pl.ds(base, sck)], dst_v)
            descs = [
                pltpu.async_copy(
                    rows_v.at[pl.ds(j * CH, CH)],
                    acc.at[dst_v.at[j]],
                    sems[j],
                    add=True,
                )
                for j in range(sck)
            ]
            for d in descs:
                d.wait()
            return carry

        lax.fori_loop(0, n_chunks_w // sck, body, 0)
        plsc.subcore_barrier()
        pltpu.sync_copy(
            acc.at[pl.ds(sid * nz, nz)], out_hbm.at[cid].at[pl.ds(sid * nz, nz)]
        )

    return kfn(m_rows, dst_chunks)


def _conv_level(children, prune, src, dst, w_blk, n_out, sck_g, sck_s,
                stage=0):
    kc, e = src.shape
    c = children.shape[1]
    cout = w_blk.shape[-1]
    sck_lcm = sck_g * sck_s // math.gcd(sck_g, sck_s)
    r = kc * e
    r_pad = _rpad(r, sck_lcm)
    n_chunks_w = r_pad // (CH * NW)
    nch = r_pad // CH

    srcf = jnp.pad(src.reshape(r), (0, r_pad - r))
    dstf = jnp.pad(dst.reshape(r), (0, r_pad - r), constant_values=n_out)

    g = _sc_gather(children, srcf, prune, n_chunks_w, sck_g, stage)

    pk = 128 // cout
    cw = pk * c
    wbd = jnp.stack([_blockdiag(w_blk[k], pk) for k in range(kc)])
    m_packed = _tc_edge_matmul(
        g.reshape(r_pad // pk, cw), wbd, e // pk, r_pad // pk
    )

    return _sc_scatter(
        m_packed.reshape(r_pad, cout), dstf.reshape(nch, CH), n_out,
        n_chunks_w, sck_s,
    )


def kernel(x, W_up1, b_up1, W_blk1, b_blk1, W_up2, b_up2, W_blk2, b_blk2,
           W_up3, b_up3, W_blk3, b_blk3, prune1_idx, prune2_idx, prune3_idx,
           src1, dst1, src2, dst2, src3, dst3):
    kup = W_up1.shape[0]

    def upw(w):
        return jnp.transpose(w, (1, 0, 2)).reshape(w.shape[1], -1)

    def upb(b):
        return jnp.tile(b, (kup,)).reshape(1, -1)

    n2, n1, n0 = prune1_idx.shape[0], prune2_idx.shape[0], prune3_idx.shape[0]

    ch1 = _tc_matmul(x, upw(W_up1), upb(b_up1))
    ch1 = ch1.reshape(-1, W_up1.shape[-1])
    p1 = _conv_level(ch1, prune1_idx, src1, dst1, W_blk1, n2, 4, 4,
                     stage=ch1.shape[0])

    ch2 = _tc_combine_matmul(
        p1, b_blk1.reshape(1, -1), upw(W_up2), upb(b_up2)
    )
    ch2 = ch2.reshape(-1, W_up2.shape[-1])
    p2 = _conv_level(ch2, prune2_idx, src2, dst2, W_blk2, n1, 1, 4,
                     stage=25088)

    n_acc1 = p2.shape[1]
    c2 = W_up3.shape[1]
    p2pk = p2.reshape(NC, n_acc1 // 2, 2 * c2)
    bi2 = jnp.tile(b_blk2, (2,)).reshape(1, -1)
    w3bd = _blockdiag(upw(W_up3), 2)
    b3t = jnp.tile(jnp.tile(b_up3, (kup,)), (2,)).reshape(1, -1)
    ch3 = _tc_combine_matmul(p2pk, bi2, w3bd, b3t)
    ch3 = ch3.reshape(-1, W_up3.shape[-1])

    cout = W_blk3.shape[-1]
    cpad = LANES
    w3p = jnp.pad(W_blk3, ((0, 0), (0, 0), (0, cpad - cout)))
    p3 = _conv_level(ch3, prune3_idx, src3, dst3, w3p, n0, 16, 16)

    n_acc0 = p3.shape[1]
    parts_r = p3.reshape(NC, n_acc0 * cpad // 128, 128)
    b3tile = jnp.tile(jnp.pad(b_blk3, (0, cpad - cout)), (128 // cpad,)).reshape(1, 128)
    outr = _tc_final_combine(parts_r, b3tile)
    return outr.reshape(n_acc0, cpad)[:n0, :cout]
